# baseline (speedup 1.0000x reference)
import functools

import jax
import jax.numpy as jnp
from jax import lax
from jax.experimental import pallas as pl
from jax.experimental.pallas import tpu as pltpu
from jax.experimental.pallas import tpu_sc as plsc

B = 16
N_SNPS = 100000
N_GENES = 18000
NF = 8
N_NODES = 300000

NC = 2
NS = 16
NW = NC * NS
ROWS_PER_DMA = 128
NBUF = 8
K0 = 64
K1 = 88
KMAX = max(K0, K1)
TOT_CHUNKS = NS * (K0 + K1)
N_PAD = ROWS_PER_DMA * TOT_CHUNKS
SENTINEL = N_GENES
G_ACC = 18048
ZROWS = G_ACC // NS
OROWS = G_ACC // NS

S_BLK = 8192


def _prep_body(snp_ref, filt_ref, out_ref):
    f = jnp.sum(filt_ref[...], axis=0) * (1.0 / NF)
    out_ref[...] = snp_ref[...] * f[None, :]


def _prep(snp, filters):
    grid = (pl.cdiv(N_SNPS, S_BLK),)
    return pl.pallas_call(
        _prep_body,
        grid=grid,
        in_specs=[
            pl.BlockSpec((B, S_BLK), lambda i: (0, i)),
            pl.BlockSpec((NF, S_BLK), lambda i: (0, i)),
        ],
        out_specs=pl.BlockSpec((B, S_BLK), lambda i: (0, i)),
        out_shape=jax.ShapeDtypeStruct((B, N_SNPS), jnp.float32),
    )(snp, filters)


def _sc_body(wsnp_hbm, ids_hbm, seg_hbm, out_hbm,
             idx_v, seg_v, rows_v, zbuf_v, acc_sh,
             isem, jsem, gsems, ssems):
    c = lax.axis_index("c")
    s = lax.axis_index("s")

    @pl.when(c == 0)
    def _():
        pltpu.async_copy(ids_hbm.at[pl.ds(s * K0, K0)],
                         idx_v.at[pl.ds(0, K0)], isem)
        pltpu.async_copy(seg_hbm.at[pl.ds(s * K0, K0)],
                         seg_v.at[pl.ds(0, K0)], jsem)

    @pl.when(c == 1)
    def _():
        pltpu.async_copy(ids_hbm.at[pl.ds(NS * K0 + s * K1, K1)],
                         idx_v.at[pl.ds(0, K1)], isem)
        pltpu.async_copy(seg_hbm.at[pl.ds(NS * K0 + s * K1, K1)],
                         seg_v.at[pl.ds(0, K1)], jsem)

    n_groups = jnp.where(c == 0, K0 // NBUF, K1 // NBUF)

    z = jnp.zeros((16,), jnp.float32)

    def _zb(i, carry):
        zbuf_v[i, :] = z
        return carry

    lax.fori_loop(0, ZROWS, _zb, 0)
    pltpu.sync_copy(zbuf_v, acc_sh.at[pl.ds(s * ZROWS, ZROWS)])

    @pl.when(c == 0)
    def _():
        pltpu.make_async_copy(ids_hbm.at[pl.ds(s * K0, K0)],
                              idx_v.at[pl.ds(0, K0)], isem).wait()
        pltpu.make_async_copy(seg_hbm.at[pl.ds(s * K0, K0)],
                              seg_v.at[pl.ds(0, K0)], jsem).wait()

    @pl.when(c == 1)
    def _():
        pltpu.make_async_copy(ids_hbm.at[pl.ds(NS * K0 + s * K1, K1)],
                              idx_v.at[pl.ds(0, K1)], isem).wait()
        pltpu.make_async_copy(seg_hbm.at[pl.ds(NS * K0 + s * K1, K1)],
                              seg_v.at[pl.ds(0, K1)], jsem).wait()

    plsc.subcore_barrier()

    def _gather(j, b):
        return pltpu.make_async_copy(wsnp_hbm.at[idx_v.at[j]], rows_v.at[b],
                                     gsems.at[b])

    def _scatter(j, b):
        return pltpu.make_async_copy(rows_v.at[b], acc_sh.at[seg_v.at[j]],
                                     ssems.at[b])

    for b in range(NBUF):
        _gather(b, b).start()

    def _group(g, carry):
        base = g * NBUF
        for b in range(NBUF):
            _gather(base + b, b).wait()
            _scatter(base + b, b).start(add=True)
        for b in range(NBUF):
            _scatter(base + b, b).wait()

            @pl.when(g + 1 < n_groups)
            def _():
                _gather(base + NBUF + b, b).start()
        return carry

    lax.fori_loop(0, n_groups, _group, 0)
    plsc.subcore_barrier()

    r0 = s * OROWS
    pltpu.sync_copy(acc_sh.at[pl.ds(r0, OROWS)],
                    out_hbm.at[pl.ds(c * G_ACC + r0, OROWS)])


_sc_kernel = functools.partial(
    pl.kernel,
    out_type=jax.ShapeDtypeStruct((NC * G_ACC, B), jnp.float32),
    mesh=plsc.VectorSubcoreMesh(core_axis_name="c", subcore_axis_name="s",
                                num_cores=NC, num_subcores=NS),
    scratch_types=[
        pltpu.VMEM((KMAX, ROWS_PER_DMA), jnp.int32),
        pltpu.VMEM((KMAX, ROWS_PER_DMA), jnp.int32),
        pltpu.VMEM((NBUF, ROWS_PER_DMA, B), jnp.float32),
        pltpu.VMEM((ZROWS, B), jnp.float32),
        pltpu.VMEM_SHARED((G_ACC, B), jnp.float32),
        pltpu.SemaphoreType.DMA,
        pltpu.SemaphoreType.DMA,
        pltpu.SemaphoreType.DMA((NBUF,)),
        pltpu.SemaphoreType.DMA((NBUF,)),
    ],
    compiler_params=pltpu.CompilerParams(use_tc_tiling_on_sc=False),
)(_sc_body)


def _mlp_body(p_ref, w1_ref, b1_ref, g1_ref, h1_ref,
              w2_ref, b2_ref, g2_ref, h2_ref, wm_ref, bm_ref, out_ref):
    psum = p_ref[0] + p_ref[1]
    inv = lax.rsqrt(jnp.float32(1.0 + 1e-5))
    h = lax.dot_general(w1_ref[...], psum, (((1,), (0,)), ((), ())),
                        preferred_element_type=jnp.float32)
    h = (h + b1_ref[...]) * inv * g1_ref[...] + h1_ref[...]
    h = jnp.maximum(h, 0.0)
    h2 = lax.dot_general(w2_ref[...], h, (((1,), (0,)), ((), ())),
                         preferred_element_type=jnp.float32)
    h2 = (h2 + b2_ref[...]) * inv * g2_ref[...] + h2_ref[...]
    feat = jnp.maximum(h2, 0.0)
    lg = lax.dot_general(wm_ref[...], feat, (((1,), (0,)), ((), ())),
                         preferred_element_type=jnp.float32)
    out_ref[...] = lg + bm_ref[...]


def _mlp(p3, W1, b1c, g1c, h1c, W2, b2c, g2c, h2c, Wm_p, bm_c):
    specs = [pl.BlockSpec((NC, N_GENES, B), lambda i: (0, 0, 0))]
    specs += [pl.BlockSpec(x.shape, lambda i, _n=len(x.shape): (0,) * _n)
              for x in (W1, b1c, g1c, h1c, W2, b2c, g2c, h2c, Wm_p, bm_c)]
    return pl.pallas_call(
        _mlp_body,
        grid=(1,),
        in_specs=specs,
        out_specs=pl.BlockSpec((1, B), lambda i: (0, 0)),
        out_shape=jax.ShapeDtypeStruct((1, B), jnp.float32),
    )(p3, W1, b1c, g1c, h1c, W2, b2c, g2c, h2c, Wm_p, bm_c)


def kernel(snp, snp_ids, segment_ids, filters, W1, b1, bn1_w, bn1_b,
           W2, b2, bn2_w, bn2_b, Wm, bm):
    wsnp = jnp.transpose(_prep(snp, filters))

    pad = N_PAD - N_NODES
    ids_p = jnp.concatenate(
        [snp_ids, jnp.zeros((pad,), jnp.int32)]).reshape(TOT_CHUNKS,
                                                         ROWS_PER_DMA)
    seg_p = jnp.concatenate(
        [segment_ids, jnp.full((pad,), SENTINEL, jnp.int32)]).reshape(
            TOT_CHUNKS, ROWS_PER_DMA)

    parts = _sc_kernel(wsnp, ids_p, seg_p)
    p3 = parts.reshape(NC, G_ACC, B)

    Wm_p = jnp.concatenate([Wm, jnp.zeros((1, 1), jnp.float32)], axis=1)
    logits = _mlp(p3, W1,
                  b1.reshape(-1, 1), bn1_w.reshape(-1, 1), bn1_b.reshape(-1, 1),
                  W2,
                  b2.reshape(-1, 1), bn2_w.reshape(-1, 1), bn2_b.reshape(-1, 1),
                  Wm_p, bm.reshape(1, 1))
    return logits.reshape(B, 1)


# speedup vs baseline: 1.4796x; 1.4796x over previous
import functools

import jax
import jax.numpy as jnp
from jax import lax
from jax.experimental import pallas as pl
from jax.experimental.pallas import tpu as pltpu
from jax.experimental.pallas import tpu_sc as plsc

B = 16
N_SNPS = 100000
N_GENES = 18000
NF = 8
N_NODES = 300000

NC = 2
NS = 16
NW = NC * NS
ROWS_PER_DMA = 128
NBUF = 8
K0 = 80
K1 = 80
KMAX = max(K0, K1)
TOT_CHUNKS = NS * (K0 + K1)
N_PAD = ROWS_PER_DMA * TOT_CHUNKS
PAD_SPREAD = 1024
G_ACC = 19072
ZROWS = G_ACC // NS
OROWS = G_ACC // NS

S_BLK = 8192


def _prep_body(snp_ref, filt_ref, out_ref):
    f = jnp.sum(filt_ref[...], axis=0) * (1.0 / NF)
    out_ref[...] = snp_ref[...] * f[None, :]


def _prep(snp, filters):
    grid = (pl.cdiv(N_SNPS, S_BLK),)
    return pl.pallas_call(
        _prep_body,
        grid=grid,
        in_specs=[
            pl.BlockSpec((B, S_BLK), lambda i: (0, i)),
            pl.BlockSpec((NF, S_BLK), lambda i: (0, i)),
        ],
        out_specs=pl.BlockSpec((B, S_BLK), lambda i: (0, i)),
        out_shape=jax.ShapeDtypeStruct((B, N_SNPS), jnp.float32),
    )(snp, filters)


def _sc_body(wsnp_hbm, ids_hbm, seg_hbm, out_hbm,
             idx_v, seg_v, rows_v, zbuf_v, acc_sh,
             isem, jsem, gsems, ssems):
    c = lax.axis_index("c")
    s = lax.axis_index("s")

    @pl.when(c == 0)
    def _():
        pltpu.async_copy(ids_hbm.at[pl.ds(s * K0, K0)],
                         idx_v.at[pl.ds(0, K0)], isem)
        pltpu.async_copy(seg_hbm.at[pl.ds(s * K0, K0)],
                         seg_v.at[pl.ds(0, K0)], jsem)

    @pl.when(c == 1)
    def _():
        pltpu.async_copy(ids_hbm.at[pl.ds(NS * K0 + s * K1, K1)],
                         idx_v.at[pl.ds(0, K1)], isem)
        pltpu.async_copy(seg_hbm.at[pl.ds(NS * K0 + s * K1, K1)],
                         seg_v.at[pl.ds(0, K1)], jsem)

    n_groups = jnp.where(c == 0, K0 // NBUF, K1 // NBUF)

    z = jnp.zeros((16,), jnp.float32)

    def _zb(i, carry):
        zbuf_v[i, :] = z
        return carry

    lax.fori_loop(0, ZROWS, _zb, 0)
    pltpu.sync_copy(zbuf_v, acc_sh.at[pl.ds(s * ZROWS, ZROWS)])

    @pl.when(c == 0)
    def _():
        pltpu.make_async_copy(ids_hbm.at[pl.ds(s * K0, K0)],
                              idx_v.at[pl.ds(0, K0)], isem).wait()
        pltpu.make_async_copy(seg_hbm.at[pl.ds(s * K0, K0)],
                              seg_v.at[pl.ds(0, K0)], jsem).wait()

    @pl.when(c == 1)
    def _():
        pltpu.make_async_copy(ids_hbm.at[pl.ds(NS * K0 + s * K1, K1)],
                              idx_v.at[pl.ds(0, K1)], isem).wait()
        pltpu.make_async_copy(seg_hbm.at[pl.ds(NS * K0 + s * K1, K1)],
                              seg_v.at[pl.ds(0, K1)], jsem).wait()

    plsc.subcore_barrier()

    def _gather(j, b):
        return pltpu.make_async_copy(wsnp_hbm.at[idx_v.at[j]], rows_v.at[b],
                                     gsems.at[b])

    def _scatter(j, b):
        return pltpu.make_async_copy(rows_v.at[b], acc_sh.at[seg_v.at[j]],
                                     ssems.at[b])

    for b in range(NBUF):
        _gather(b, b).start()

    def _group(g, carry):
        base = g * NBUF
        for b in range(NBUF):
            _gather(base + b, b).wait()
            _scatter(base + b, b).start(add=True)
        for b in range(NBUF):
            _scatter(base + b, b).wait()

            @pl.when(g + 1 < n_groups)
            def _():
                _gather(base + NBUF + b, b).start()
        return carry

    lax.fori_loop(0, n_groups, _group, 0)
    plsc.subcore_barrier()

    r0 = s * OROWS
    pltpu.sync_copy(acc_sh.at[pl.ds(r0, OROWS)],
                    out_hbm.at[pl.ds(c * G_ACC + r0, OROWS)])


_sc_kernel = functools.partial(
    pl.kernel,
    out_type=jax.ShapeDtypeStruct((NC * G_ACC, B), jnp.float32),
    mesh=plsc.VectorSubcoreMesh(core_axis_name="c", subcore_axis_name="s",
                                num_cores=NC, num_subcores=NS),
    scratch_types=[
        pltpu.VMEM((KMAX, ROWS_PER_DMA), jnp.int32),
        pltpu.VMEM((KMAX, ROWS_PER_DMA), jnp.int32),
        pltpu.VMEM((NBUF, ROWS_PER_DMA, B), jnp.float32),
        pltpu.VMEM((ZROWS, B), jnp.float32),
        pltpu.VMEM_SHARED((G_ACC, B), jnp.float32),
        pltpu.SemaphoreType.DMA,
        pltpu.SemaphoreType.DMA,
        pltpu.SemaphoreType.DMA((NBUF,)),
        pltpu.SemaphoreType.DMA((NBUF,)),
    ],
    compiler_params=pltpu.CompilerParams(use_tc_tiling_on_sc=False),
)(_sc_body)


def _mlp_body(p_ref, w1_ref, b1_ref, g1_ref, h1_ref,
              w2_ref, b2_ref, g2_ref, h2_ref, wm_ref, bm_ref, out_ref):
    psum = p_ref[0] + p_ref[1]
    inv = lax.rsqrt(jnp.float32(1.0 + 1e-5))
    h = lax.dot_general(w1_ref[...], psum, (((1,), (0,)), ((), ())),
                        preferred_element_type=jnp.float32)
    h = (h + b1_ref[...]) * inv * g1_ref[...] + h1_ref[...]
    h = jnp.maximum(h, 0.0)
    h2 = lax.dot_general(w2_ref[...], h, (((1,), (0,)), ((), ())),
                         preferred_element_type=jnp.float32)
    h2 = (h2 + b2_ref[...]) * inv * g2_ref[...] + h2_ref[...]
    feat = jnp.maximum(h2, 0.0)
    lg = lax.dot_general(wm_ref[...], feat, (((1,), (0,)), ((), ())),
                         preferred_element_type=jnp.float32)
    out_ref[...] = lg + bm_ref[...]


def _mlp(p3, W1, b1c, g1c, h1c, W2, b2c, g2c, h2c, Wm_p, bm_c):
    specs = [pl.BlockSpec((NC, N_GENES, B), lambda i: (0, 0, 0))]
    specs += [pl.BlockSpec(x.shape, lambda i, _n=len(x.shape): (0,) * _n)
              for x in (W1, b1c, g1c, h1c, W2, b2c, g2c, h2c, Wm_p, bm_c)]
    return pl.pallas_call(
        _mlp_body,
        grid=(1,),
        in_specs=specs,
        out_specs=pl.BlockSpec((1, B), lambda i: (0, 0)),
        out_shape=jax.ShapeDtypeStruct((1, B), jnp.float32),
    )(p3, W1, b1c, g1c, h1c, W2, b2c, g2c, h2c, Wm_p, bm_c)


def kernel(snp, snp_ids, segment_ids, filters, W1, b1, bn1_w, bn1_b,
           W2, b2, bn2_w, bn2_b, Wm, bm):
    wsnp = jnp.transpose(_prep(snp, filters))

    pad = N_PAD - N_NODES
    it = jnp.arange(pad, dtype=jnp.int32)
    ids_p = jnp.concatenate(
        [snp_ids, it % N_SNPS]).reshape(TOT_CHUNKS, ROWS_PER_DMA)
    seg_p = jnp.concatenate(
        [segment_ids, N_GENES + it % PAD_SPREAD]).reshape(TOT_CHUNKS,
                                                          ROWS_PER_DMA)

    parts = _sc_kernel(wsnp, ids_p, seg_p)
    p3 = parts.reshape(NC, G_ACC, B)

    Wm_p = jnp.concatenate([Wm, jnp.zeros((1, 1), jnp.float32)], axis=1)
    logits = _mlp(p3, W1,
                  b1.reshape(-1, 1), bn1_w.reshape(-1, 1), bn1_b.reshape(-1, 1),
                  W2,
                  b2.reshape(-1, 1), bn2_w.reshape(-1, 1), bn2_b.reshape(-1, 1),
                  Wm_p, bm.reshape(1, 1))
    return logits.reshape(B, 1)


# speedup vs baseline: 1.5788x; 1.0670x over previous
import functools

import jax
import jax.numpy as jnp
from jax import lax
from jax.experimental import pallas as pl
from jax.experimental.pallas import tpu as pltpu
from jax.experimental.pallas import tpu_sc as plsc

B = 16
N_SNPS = 100000
N_GENES = 18000
NF = 8
N_NODES = 300000

NC = 2
NS = 16
NW = NC * NS
ROWS_PER_DMA = 128
NBUF = 8
K0 = 80
K1 = 80
KMAX = max(K0, K1)
TOT_CHUNKS = NS * (K0 + K1)
N_PAD = ROWS_PER_DMA * TOT_CHUNKS
PAD_SPREAD = 1024
G_ACC = 19072
ZROWS = G_ACC // NS
OROWS = G_ACC // NS

SPAN = 3136
SPAN_LAST = N_SNPS - 31 * SPAN
HSPAN = SPAN // 2
HSPAN_LAST = SPAN_LAST // 2


def _tr_body(snp_hbm, filt_hbm, tbl_hbm, cbuf, fbuf, fbar_v, obuf, sem):
    c = lax.axis_index("c")
    s = lax.axis_index("s")
    t = s * NC + c
    iota = lax.iota(jnp.int32, B)

    def _do(col0, span, hspan):
        descs = [pltpu.async_copy(snp_hbm.at[b, pl.ds(col0, span)],
                                  cbuf.at[b, pl.ds(0, span)], sem)
                 for b in range(B)]
        for h in range(2):
            fd = [pltpu.async_copy(
                filt_hbm.at[k, pl.ds(col0 + h * hspan, hspan)],
                fbuf.at[k, pl.ds(0, hspan)], sem) for k in range(NF)]
            for d in fd:
                d.wait()

            def _fb(u, carry):
                off = u * B
                acc = fbuf[0, pl.ds(off, B)]
                for k in range(1, NF):
                    acc = acc + fbuf[k, pl.ds(off, B)]
                fbar_v[pl.ds(h * hspan + off, B)] = acc * (1.0 / NF)
                return carry

            lax.fori_loop(0, hspan // B, _fb, 0)
        for d in descs:
            d.wait()

        def _colgrp(u, carry):
            off = u * B
            fv = fbar_v[pl.ds(off, B)]
            for k in range(B):
                j = off + k
                vals = plsc.load_gather(
                    cbuf, [iota, jnp.full((B,), j, jnp.int32)])
                obuf[j, :] = vals * fv[k]
            return carry

        lax.fori_loop(0, span // B, _colgrp, 0)
        pltpu.sync_copy(obuf.at[pl.ds(0, span)],
                        tbl_hbm.at[pl.ds(col0, span)])

    @pl.when(t < 31)
    def _():
        _do(t * SPAN, SPAN, HSPAN)

    @pl.when(t == 31)
    def _():
        _do(31 * SPAN, SPAN_LAST, HSPAN_LAST)


_sc_transpose = functools.partial(
    pl.kernel,
    out_type=jax.ShapeDtypeStruct((N_SNPS, B), jnp.float32),
    mesh=plsc.VectorSubcoreMesh(core_axis_name="c", subcore_axis_name="s",
                                num_cores=NC, num_subcores=NS),
    scratch_types=[
        pltpu.VMEM((B, SPAN), jnp.float32),
        pltpu.VMEM((NF, HSPAN), jnp.float32),
        pltpu.VMEM((SPAN,), jnp.float32),
        pltpu.VMEM((SPAN, B), jnp.float32),
        pltpu.SemaphoreType.DMA,
    ],
    compiler_params=pltpu.CompilerParams(use_tc_tiling_on_sc=False,
                                         needs_layout_passes=False),
)(_tr_body)


def _sc_body(wsnp_hbm, ids_hbm, seg_hbm, out_hbm,
             idx_v, seg_v, rows_v, zbuf_v, acc_sh,
             isem, jsem, gsems, ssems):
    c = lax.axis_index("c")
    s = lax.axis_index("s")

    @pl.when(c == 0)
    def _():
        pltpu.async_copy(ids_hbm.at[pl.ds(s * K0, K0)],
                         idx_v.at[pl.ds(0, K0)], isem)
        pltpu.async_copy(seg_hbm.at[pl.ds(s * K0, K0)],
                         seg_v.at[pl.ds(0, K0)], jsem)

    @pl.when(c == 1)
    def _():
        pltpu.async_copy(ids_hbm.at[pl.ds(NS * K0 + s * K1, K1)],
                         idx_v.at[pl.ds(0, K1)], isem)
        pltpu.async_copy(seg_hbm.at[pl.ds(NS * K0 + s * K1, K1)],
                         seg_v.at[pl.ds(0, K1)], jsem)

    n_groups = jnp.where(c == 0, K0 // NBUF, K1 // NBUF)

    z = jnp.zeros((16,), jnp.float32)

    def _zb(i, carry):
        zbuf_v[i, :] = z
        return carry

    lax.fori_loop(0, ZROWS, _zb, 0)
    pltpu.sync_copy(zbuf_v, acc_sh.at[pl.ds(s * ZROWS, ZROWS)])

    @pl.when(c == 0)
    def _():
        pltpu.make_async_copy(ids_hbm.at[pl.ds(s * K0, K0)],
                              idx_v.at[pl.ds(0, K0)], isem).wait()
        pltpu.make_async_copy(seg_hbm.at[pl.ds(s * K0, K0)],
                              seg_v.at[pl.ds(0, K0)], jsem).wait()

    @pl.when(c == 1)
    def _():
        pltpu.make_async_copy(ids_hbm.at[pl.ds(NS * K0 + s * K1, K1)],
                              idx_v.at[pl.ds(0, K1)], isem).wait()
        pltpu.make_async_copy(seg_hbm.at[pl.ds(NS * K0 + s * K1, K1)],
                              seg_v.at[pl.ds(0, K1)], jsem).wait()

    plsc.subcore_barrier()

    def _gather(j, b):
        return pltpu.make_async_copy(wsnp_hbm.at[idx_v.at[j]], rows_v.at[b],
                                     gsems.at[b])

    def _scatter(j, b):
        return pltpu.make_async_copy(rows_v.at[b], acc_sh.at[seg_v.at[j]],
                                     ssems.at[b])

    for b in range(NBUF):
        _gather(b, b).start()

    def _group(g, carry):
        base = g * NBUF
        for b in range(NBUF):
            _gather(base + b, b).wait()
            _scatter(base + b, b).start(add=True)
        for b in range(NBUF):
            _scatter(base + b, b).wait()

            @pl.when(g + 1 < n_groups)
            def _():
                _gather(base + NBUF + b, b).start()
        return carry

    lax.fori_loop(0, n_groups, _group, 0)
    plsc.subcore_barrier()

    r0 = s * OROWS
    pltpu.sync_copy(acc_sh.at[pl.ds(r0, OROWS)],
                    out_hbm.at[pl.ds(c * G_ACC + r0, OROWS)])


_sc_kernel = functools.partial(
    pl.kernel,
    out_type=jax.ShapeDtypeStruct((NC * G_ACC, B), jnp.float32),
    mesh=plsc.VectorSubcoreMesh(core_axis_name="c", subcore_axis_name="s",
                                num_cores=NC, num_subcores=NS),
    scratch_types=[
        pltpu.VMEM((KMAX, ROWS_PER_DMA), jnp.int32),
        pltpu.VMEM((KMAX, ROWS_PER_DMA), jnp.int32),
        pltpu.VMEM((NBUF, ROWS_PER_DMA, B), jnp.float32),
        pltpu.VMEM((ZROWS, B), jnp.float32),
        pltpu.VMEM_SHARED((G_ACC, B), jnp.float32),
        pltpu.SemaphoreType.DMA,
        pltpu.SemaphoreType.DMA,
        pltpu.SemaphoreType.DMA((NBUF,)),
        pltpu.SemaphoreType.DMA((NBUF,)),
    ],
    compiler_params=pltpu.CompilerParams(use_tc_tiling_on_sc=False),
)(_sc_body)


def _mlp_body(p_ref, w1_ref, b1_ref, g1_ref, h1_ref,
              w2_ref, b2_ref, g2_ref, h2_ref, wm_ref, bm_ref, out_ref):
    psum = p_ref[0] + p_ref[1]
    inv = lax.rsqrt(jnp.float32(1.0 + 1e-5))
    h = lax.dot_general(w1_ref[...], psum, (((1,), (0,)), ((), ())),
                        preferred_element_type=jnp.float32)
    h = (h + b1_ref[...]) * inv * g1_ref[...] + h1_ref[...]
    h = jnp.maximum(h, 0.0)
    h2 = lax.dot_general(w2_ref[...], h, (((1,), (0,)), ((), ())),
                         preferred_element_type=jnp.float32)
    h2 = (h2 + b2_ref[...]) * inv * g2_ref[...] + h2_ref[...]
    feat = jnp.maximum(h2, 0.0)
    lg = lax.dot_general(wm_ref[...], feat, (((1,), (0,)), ((), ())),
                         preferred_element_type=jnp.float32)
    out_ref[...] = lg + bm_ref[...]


def _mlp(p3, W1, b1c, g1c, h1c, W2, b2c, g2c, h2c, Wm_p, bm_c):
    specs = [pl.BlockSpec((NC, N_GENES, B), lambda i: (0, 0, 0))]
    specs += [pl.BlockSpec(x.shape, lambda i, _n=len(x.shape): (0,) * _n)
              for x in (W1, b1c, g1c, h1c, W2, b2c, g2c, h2c, Wm_p, bm_c)]
    return pl.pallas_call(
        _mlp_body,
        grid=(1,),
        in_specs=specs,
        out_specs=pl.BlockSpec((1, B), lambda i: (0, 0)),
        out_shape=jax.ShapeDtypeStruct((1, B), jnp.float32),
    )(p3, W1, b1c, g1c, h1c, W2, b2c, g2c, h2c, Wm_p, bm_c)


def kernel(snp, snp_ids, segment_ids, filters, W1, b1, bn1_w, bn1_b,
           W2, b2, bn2_w, bn2_b, Wm, bm):
    wsnp = _sc_transpose(snp, filters)

    pad = N_PAD - N_NODES
    it = jnp.arange(pad, dtype=jnp.int32)
    ids_p = jnp.concatenate(
        [snp_ids, it % N_SNPS]).reshape(TOT_CHUNKS, ROWS_PER_DMA)
    seg_p = jnp.concatenate(
        [segment_ids, N_GENES + it % PAD_SPREAD]).reshape(TOT_CHUNKS,
                                                          ROWS_PER_DMA)

    parts = _sc_kernel(wsnp, ids_p, seg_p)
    p3 = parts.reshape(NC, G_ACC, B)

    Wm_p = jnp.concatenate([Wm, jnp.zeros((1, 1), jnp.float32)], axis=1)
    logits = _mlp(p3, W1,
                  b1.reshape(-1, 1), bn1_w.reshape(-1, 1), bn1_b.reshape(-1, 1),
                  W2,
                  b2.reshape(-1, 1), bn2_w.reshape(-1, 1), bn2_b.reshape(-1, 1),
                  Wm_p, bm.reshape(1, 1))
    return logits.reshape(B, 1)


# speedup vs baseline: 1.8766x; 1.1886x over previous
import functools

import jax
import jax.numpy as jnp
from jax import lax
from jax.experimental import pallas as pl
from jax.experimental.pallas import tpu as pltpu
from jax.experimental.pallas import tpu_sc as plsc

B = 16
N_SNPS = 100000
N_GENES = 18000
NF = 8
N_NODES = 300000

NC = 2
NS = 16
NW = NC * NS
ROWS_PER_DMA = 128
NBUF = 8
K0 = 80
K1 = 80
KMAX = max(K0, K1)
TOT_CHUNKS = NS * (K0 + K1)
N_PAD = ROWS_PER_DMA * TOT_CHUNKS
PAD_SPREAD = 1024
G_ACC = 19072
ZROWS = G_ACC // NS
OROWS = G_ACC // NS

SPAN = 3136
SPAN_LAST = N_SNPS - 31 * SPAN
HSPAN = SPAN // 2
HSPAN_LAST = SPAN_LAST // 2


def _tr_body(snp_hbm, filt_hbm, tbl_hbm, cbuf, fbuf, fbar_v, obuf, sem):
    c = lax.axis_index("c")
    s = lax.axis_index("s")
    t = s * NC + c
    iota = lax.iota(jnp.int32, B)

    def _do(col0, span, hspan):
        descs = [pltpu.async_copy(snp_hbm.at[b, pl.ds(col0, span)],
                                  cbuf.at[pl.ds(b * SPAN, span)], sem)
                 for b in range(B)]
        for h in range(2):
            fd = [pltpu.async_copy(
                filt_hbm.at[k, pl.ds(col0 + h * hspan, hspan)],
                fbuf.at[k, pl.ds(0, hspan)], sem) for k in range(NF)]
            for d in fd:
                d.wait()

            def _fb(u, carry):
                off = u * B
                acc = fbuf[0, pl.ds(off, B)]
                for k in range(1, NF):
                    acc = acc + fbuf[k, pl.ds(off, B)]
                fbar_v[pl.ds(h * hspan + off, B)] = acc * (1.0 / NF)
                return carry

            lax.fori_loop(0, hspan // B, _fb, 0)
        for d in descs:
            d.wait()

        @plsc.parallel_loop(0, span // B, 1, unroll=2)
        def _ps(u):
            off = u * B
            fv = fbar_v[pl.ds(off, B)]
            for b in range(B):
                cbuf[pl.ds(b * SPAN + off, B)] = (
                    cbuf[pl.ds(b * SPAN + off, B)] * fv)

        base = iota * SPAN

        @plsc.parallel_loop(0, span, 1, unroll=8)
        def _col(j):
            obuf[j, :] = plsc.load_gather(cbuf, [base + j])
        pltpu.sync_copy(obuf.at[pl.ds(0, span)],
                        tbl_hbm.at[pl.ds(col0, span)])

    @pl.when(t < 31)
    def _():
        _do(t * SPAN, SPAN, HSPAN)

    @pl.when(t == 31)
    def _():
        _do(31 * SPAN, SPAN_LAST, HSPAN_LAST)


_sc_transpose = functools.partial(
    pl.kernel,
    out_type=jax.ShapeDtypeStruct((N_SNPS, B), jnp.float32),
    mesh=plsc.VectorSubcoreMesh(core_axis_name="c", subcore_axis_name="s",
                                num_cores=NC, num_subcores=NS),
    scratch_types=[
        pltpu.VMEM((B * SPAN,), jnp.float32),
        pltpu.VMEM((NF, HSPAN), jnp.float32),
        pltpu.VMEM((SPAN,), jnp.float32),
        pltpu.VMEM((SPAN, B), jnp.float32),
        pltpu.SemaphoreType.DMA,
    ],
    compiler_params=pltpu.CompilerParams(use_tc_tiling_on_sc=False,
                                         needs_layout_passes=False),
)(_tr_body)


def _sc_body(wsnp_hbm, ids_hbm, seg_hbm, out_hbm,
             idx_v, seg_v, rows_v, zbuf_v, acc_sh,
             isem, jsem, gsems, ssems):
    c = lax.axis_index("c")
    s = lax.axis_index("s")

    @pl.when(c == 0)
    def _():
        pltpu.async_copy(ids_hbm.at[pl.ds(s * K0, K0)],
                         idx_v.at[pl.ds(0, K0)], isem)
        pltpu.async_copy(seg_hbm.at[pl.ds(s * K0, K0)],
                         seg_v.at[pl.ds(0, K0)], jsem)

    @pl.when(c == 1)
    def _():
        pltpu.async_copy(ids_hbm.at[pl.ds(NS * K0 + s * K1, K1)],
                         idx_v.at[pl.ds(0, K1)], isem)
        pltpu.async_copy(seg_hbm.at[pl.ds(NS * K0 + s * K1, K1)],
                         seg_v.at[pl.ds(0, K1)], jsem)

    n_groups = jnp.where(c == 0, K0 // NBUF, K1 // NBUF)

    z = jnp.zeros((16,), jnp.float32)

    def _zb(i, carry):
        zbuf_v[i, :] = z
        return carry

    lax.fori_loop(0, ZROWS, _zb, 0)
    pltpu.sync_copy(zbuf_v, acc_sh.at[pl.ds(s * ZROWS, ZROWS)])

    @pl.when(c == 0)
    def _():
        pltpu.make_async_copy(ids_hbm.at[pl.ds(s * K0, K0)],
                              idx_v.at[pl.ds(0, K0)], isem).wait()
        pltpu.make_async_copy(seg_hbm.at[pl.ds(s * K0, K0)],
                              seg_v.at[pl.ds(0, K0)], jsem).wait()

    @pl.when(c == 1)
    def _():
        pltpu.make_async_copy(ids_hbm.at[pl.ds(NS * K0 + s * K1, K1)],
                              idx_v.at[pl.ds(0, K1)], isem).wait()
        pltpu.make_async_copy(seg_hbm.at[pl.ds(NS * K0 + s * K1, K1)],
                              seg_v.at[pl.ds(0, K1)], jsem).wait()

    plsc.subcore_barrier()

    def _gather(j, b):
        return pltpu.make_async_copy(wsnp_hbm.at[idx_v.at[j]], rows_v.at[b],
                                     gsems.at[b])

    def _scatter(j, b):
        return pltpu.make_async_copy(rows_v.at[b], acc_sh.at[seg_v.at[j]],
                                     ssems.at[b])

    for b in range(NBUF):
        _gather(b, b).start()

    def _group(g, carry):
        base = g * NBUF
        for b in range(NBUF):
            _gather(base + b, b).wait()
            _scatter(base + b, b).start(add=True)
        for b in range(NBUF):
            _scatter(base + b, b).wait()

            @pl.when(g + 1 < n_groups)
            def _():
                _gather(base + NBUF + b, b).start()
        return carry

    lax.fori_loop(0, n_groups, _group, 0)
    plsc.subcore_barrier()

    r0 = s * OROWS
    pltpu.sync_copy(acc_sh.at[pl.ds(r0, OROWS)],
                    out_hbm.at[pl.ds(c * G_ACC + r0, OROWS)])


_sc_kernel = functools.partial(
    pl.kernel,
    out_type=jax.ShapeDtypeStruct((NC * G_ACC, B), jnp.float32),
    mesh=plsc.VectorSubcoreMesh(core_axis_name="c", subcore_axis_name="s",
                                num_cores=NC, num_subcores=NS),
    scratch_types=[
        pltpu.VMEM((KMAX, ROWS_PER_DMA), jnp.int32),
        pltpu.VMEM((KMAX, ROWS_PER_DMA), jnp.int32),
        pltpu.VMEM((NBUF, ROWS_PER_DMA, B), jnp.float32),
        pltpu.VMEM((ZROWS, B), jnp.float32),
        pltpu.VMEM_SHARED((G_ACC, B), jnp.float32),
        pltpu.SemaphoreType.DMA,
        pltpu.SemaphoreType.DMA,
        pltpu.SemaphoreType.DMA((NBUF,)),
        pltpu.SemaphoreType.DMA((NBUF,)),
    ],
    compiler_params=pltpu.CompilerParams(use_tc_tiling_on_sc=False),
)(_sc_body)


def _mlp_body(p_ref, w1_ref, b1_ref, g1_ref, h1_ref,
              w2_ref, b2_ref, g2_ref, h2_ref, wm_ref, bm_ref, out_ref):
    psum = p_ref[0] + p_ref[1]
    inv = lax.rsqrt(jnp.float32(1.0 + 1e-5))
    h = lax.dot_general(w1_ref[...], psum, (((1,), (0,)), ((), ())),
                        preferred_element_type=jnp.float32)
    h = (h + b1_ref[...]) * inv * g1_ref[...] + h1_ref[...]
    h = jnp.maximum(h, 0.0)
    h2 = lax.dot_general(w2_ref[...], h, (((1,), (0,)), ((), ())),
                         preferred_element_type=jnp.float32)
    h2 = (h2 + b2_ref[...]) * inv * g2_ref[...] + h2_ref[...]
    feat = jnp.maximum(h2, 0.0)
    lg = lax.dot_general(wm_ref[...], feat, (((1,), (0,)), ((), ())),
                         preferred_element_type=jnp.float32)
    out_ref[...] = lg + bm_ref[...]


def _mlp(p3, W1, b1c, g1c, h1c, W2, b2c, g2c, h2c, Wm_p, bm_c):
    specs = [pl.BlockSpec((NC, N_GENES, B), lambda i: (0, 0, 0))]
    specs += [pl.BlockSpec(x.shape, lambda i, _n=len(x.shape): (0,) * _n)
              for x in (W1, b1c, g1c, h1c, W2, b2c, g2c, h2c, Wm_p, bm_c)]
    return pl.pallas_call(
        _mlp_body,
        grid=(1,),
        in_specs=specs,
        out_specs=pl.BlockSpec((1, B), lambda i: (0, 0)),
        out_shape=jax.ShapeDtypeStruct((1, B), jnp.float32),
    )(p3, W1, b1c, g1c, h1c, W2, b2c, g2c, h2c, Wm_p, bm_c)


def kernel(snp, snp_ids, segment_ids, filters, W1, b1, bn1_w, bn1_b,
           W2, b2, bn2_w, bn2_b, Wm, bm):
    wsnp = _sc_transpose(snp, filters)

    pad = N_PAD - N_NODES
    it = jnp.arange(pad, dtype=jnp.int32)
    ids_p = jnp.concatenate(
        [snp_ids, it % N_SNPS]).reshape(TOT_CHUNKS, ROWS_PER_DMA)
    seg_p = jnp.concatenate(
        [segment_ids, N_GENES + it % PAD_SPREAD]).reshape(TOT_CHUNKS,
                                                          ROWS_PER_DMA)

    parts = _sc_kernel(wsnp, ids_p, seg_p)
    p3 = parts.reshape(NC, G_ACC, B)

    Wm_p = jnp.concatenate([Wm, jnp.zeros((1, 1), jnp.float32)], axis=1)
    logits = _mlp(p3, W1,
                  b1.reshape(-1, 1), bn1_w.reshape(-1, 1), bn1_b.reshape(-1, 1),
                  W2,
                  b2.reshape(-1, 1), bn2_w.reshape(-1, 1), bn2_b.reshape(-1, 1),
                  Wm_p, bm.reshape(1, 1))
    return logits.reshape(B, 1)


# speedup vs baseline: 2.1396x; 1.1401x over previous
import functools

import jax
import jax.numpy as jnp
from jax import lax
from jax.experimental import pallas as pl
from jax.experimental.pallas import tpu as pltpu
from jax.experimental.pallas import tpu_sc as plsc

B = 16
N_SNPS = 100000
N_GENES = 18000
NF = 8
N_NODES = 300000

NC = 2
NS = 16
NW = NC * NS
ROWS_PER_DMA = 128
NBUF = 8
K0 = 80
K1 = 80
KMAX = max(K0, K1)
TOT_CHUNKS = NS * (K0 + K1)
N_PAD = ROWS_PER_DMA * TOT_CHUNKS
PAD_SPREAD = 1024
G_ACC = 19072
ZROWS = G_ACC // NS
OROWS = G_ACC // NS

SPAN = 3136
SPAN_LAST = N_SNPS - 31 * SPAN
HSPAN = SPAN // 2
HSPAN_LAST = SPAN_LAST // 2


def _tr_body(snp_hbm, filt_hbm, tbl_hbm, cbuf, fbuf, fbar_v, obuf, sem):
    c = lax.axis_index("c")
    s = lax.axis_index("s")
    t = s * NC + c
    iota = lax.iota(jnp.int32, B)

    def _do(col0, span, hspan):
        descs = [pltpu.async_copy(snp_hbm.at[b, pl.ds(col0, span)],
                                  cbuf.at[pl.ds(b * SPAN, span)], sem)
                 for b in range(B)]
        for h in range(2):
            fd = [pltpu.async_copy(
                filt_hbm.at[k, pl.ds(col0 + h * hspan, hspan)],
                fbuf.at[k, pl.ds(0, hspan)], sem) for k in range(NF)]
            for d in fd:
                d.wait()

            def _fb(u, carry):
                off = u * B
                acc = fbuf[0, pl.ds(off, B)]
                for k in range(1, NF):
                    acc = acc + fbuf[k, pl.ds(off, B)]
                fbar_v[pl.ds(h * hspan + off, B)] = acc * (1.0 / NF)
                return carry

            lax.fori_loop(0, hspan // B, _fb, 0)
        for d in descs:
            d.wait()

        @plsc.parallel_loop(0, span // B, 1, unroll=2)
        def _ps(u):
            off = u * B
            fv = fbar_v[pl.ds(off, B)]
            for b in range(B):
                cbuf[pl.ds(b * SPAN + off, B)] = (
                    cbuf[pl.ds(b * SPAN + off, B)] * fv)

        base = iota * SPAN

        @plsc.parallel_loop(0, span, 1, unroll=8)
        def _col(j):
            obuf[j, :] = plsc.load_gather(cbuf, [base + j])
        pltpu.sync_copy(obuf.at[pl.ds(0, span)],
                        tbl_hbm.at[pl.ds(col0, span)])

    @pl.when(t < 31)
    def _():
        _do(t * SPAN, SPAN, HSPAN)

    @pl.when(t == 31)
    def _():
        _do(31 * SPAN, SPAN_LAST, HSPAN_LAST)


_sc_transpose = functools.partial(
    pl.kernel,
    out_type=jax.ShapeDtypeStruct((N_SNPS, B), jnp.float32),
    mesh=plsc.VectorSubcoreMesh(core_axis_name="c", subcore_axis_name="s",
                                num_cores=NC, num_subcores=NS),
    scratch_types=[
        pltpu.VMEM((B * SPAN,), jnp.float32),
        pltpu.VMEM((NF, HSPAN), jnp.float32),
        pltpu.VMEM((SPAN,), jnp.float32),
        pltpu.VMEM((SPAN, B), jnp.float32),
        pltpu.SemaphoreType.DMA,
    ],
    compiler_params=pltpu.CompilerParams(use_tc_tiling_on_sc=False,
                                         needs_layout_passes=False),
)(_tr_body)


def _sc_body(wsnp_hbm, ids_hbm, seg_hbm, out_hbm,
             idx_v, seg_v, rows_v, zbuf_v, acc_sh,
             isem, jsem, gsems, ssems):
    c = lax.axis_index("c")
    s = lax.axis_index("s")

    @pl.when(c == 0)
    def _():
        pltpu.async_copy(ids_hbm.at[pl.ds(s * K0, K0)],
                         idx_v.at[pl.ds(0, K0)], isem)
        pltpu.async_copy(seg_hbm.at[pl.ds(s * K0, K0)],
                         seg_v.at[pl.ds(0, K0)], jsem)

    @pl.when(c == 1)
    def _():
        pltpu.async_copy(ids_hbm.at[pl.ds(NS * K0 + s * K1, K1)],
                         idx_v.at[pl.ds(0, K1)], isem)
        pltpu.async_copy(seg_hbm.at[pl.ds(NS * K0 + s * K1, K1)],
                         seg_v.at[pl.ds(0, K1)], jsem)

    n_groups = jnp.where(c == 0, K0 // NBUF, K1 // NBUF)

    z = jnp.zeros((16,), jnp.float32)

    def _zb(i, carry):
        zbuf_v[i, :] = z
        return carry

    lax.fori_loop(0, ZROWS, _zb, 0)
    pltpu.sync_copy(zbuf_v, acc_sh.at[pl.ds(s * ZROWS, ZROWS)])

    @pl.when(c == 0)
    def _():
        pltpu.make_async_copy(ids_hbm.at[pl.ds(s * K0, K0)],
                              idx_v.at[pl.ds(0, K0)], isem).wait()
        pltpu.make_async_copy(seg_hbm.at[pl.ds(s * K0, K0)],
                              seg_v.at[pl.ds(0, K0)], jsem).wait()

    @pl.when(c == 1)
    def _():
        pltpu.make_async_copy(ids_hbm.at[pl.ds(NS * K0 + s * K1, K1)],
                              idx_v.at[pl.ds(0, K1)], isem).wait()
        pltpu.make_async_copy(seg_hbm.at[pl.ds(NS * K0 + s * K1, K1)],
                              seg_v.at[pl.ds(0, K1)], jsem).wait()

    plsc.subcore_barrier()

    def _gather(j, b):
        return pltpu.make_async_copy(wsnp_hbm.at[idx_v.at[j]], rows_v.at[b],
                                     gsems.at[b])

    def _scatter(j, b):
        return pltpu.make_async_copy(rows_v.at[b], acc_sh.at[seg_v.at[j]],
                                     ssems.at[b])

    for b in range(NBUF):
        _gather(b, b).start()

    def _group(g, carry):
        base = g * NBUF
        for b in range(NBUF):
            _gather(base + b, b).wait()
            _scatter(base + b, b).start(add=True)
        for b in range(NBUF):
            _scatter(base + b, b).wait()

            @pl.when(g + 1 < n_groups)
            def _():
                _gather(base + NBUF + b, b).start()
        return carry

    lax.fori_loop(0, n_groups, _group, 0)
    plsc.subcore_barrier()

    r0 = s * OROWS
    pltpu.sync_copy(acc_sh.at[pl.ds(r0, OROWS)],
                    out_hbm.at[pl.ds(c * G_ACC + r0, OROWS)])


_sc_kernel = functools.partial(
    pl.kernel,
    out_type=jax.ShapeDtypeStruct((NC * G_ACC, B), jnp.float32),
    mesh=plsc.VectorSubcoreMesh(core_axis_name="c", subcore_axis_name="s",
                                num_cores=NC, num_subcores=NS),
    scratch_types=[
        pltpu.VMEM((KMAX, ROWS_PER_DMA), jnp.int32),
        pltpu.VMEM((KMAX, ROWS_PER_DMA), jnp.int32),
        pltpu.VMEM((NBUF, ROWS_PER_DMA, B), jnp.float32),
        pltpu.VMEM((ZROWS, B), jnp.float32),
        pltpu.VMEM_SHARED((G_ACC, B), jnp.float32),
        pltpu.SemaphoreType.DMA,
        pltpu.SemaphoreType.DMA,
        pltpu.SemaphoreType.DMA((NBUF,)),
        pltpu.SemaphoreType.DMA((NBUF,)),
    ],
    compiler_params=pltpu.CompilerParams(use_tc_tiling_on_sc=False),
)(_sc_body)


def _mlp_body(p_ref, w1_ref, b1_ref, g1_ref, h1_ref,
              w2_ref, b2_ref, g2_ref, h2_ref, wm_ref, bm_ref, out_ref):
    psum = p_ref[0] + p_ref[1]
    inv = lax.rsqrt(jnp.float32(1.0 + 1e-5))
    h = lax.dot_general(w1_ref[0], psum[:, 0:B], (((1,), (0,)), ((), ())),
                        preferred_element_type=jnp.float32)
    for k in range(1, 8):
        h = h + lax.dot_general(w1_ref[k], psum[:, k * B:(k + 1) * B],
                                (((1,), (0,)), ((), ())),
                                preferred_element_type=jnp.float32)
    h = (h + b1_ref[...]) * inv * g1_ref[...] + h1_ref[...]
    h = jnp.maximum(h, 0.0)
    h2 = lax.dot_general(w2_ref[...], h, (((1,), (0,)), ((), ())),
                         preferred_element_type=jnp.float32)
    h2 = (h2 + b2_ref[...]) * inv * g2_ref[...] + h2_ref[...]
    feat = jnp.maximum(h2, 0.0)
    lg = lax.dot_general(wm_ref[...], feat, (((1,), (0,)), ((), ())),
                         preferred_element_type=jnp.float32)
    out_ref[...] = lg + bm_ref[...]


def _mlp(p3, W1r, b1c, g1c, h1c, W2, b2c, g2c, h2c, Wm_p, bm_c):
    specs = [pl.BlockSpec(p3.shape, lambda i: (0, 0, 0))]
    specs += [pl.BlockSpec(x.shape, lambda i, _n=len(x.shape): (0,) * _n)
              for x in (W1r, b1c, g1c, h1c, W2, b2c, g2c, h2c, Wm_p, bm_c)]
    return pl.pallas_call(
        _mlp_body,
        grid=(1,),
        in_specs=specs,
        out_specs=pl.BlockSpec((1, B), lambda i: (0, 0)),
        out_shape=jax.ShapeDtypeStruct((1, B), jnp.float32),
    )(p3, W1r, b1c, g1c, h1c, W2, b2c, g2c, h2c, Wm_p, bm_c)


def kernel(snp, snp_ids, segment_ids, filters, W1, b1, bn1_w, bn1_b,
           W2, b2, bn2_w, bn2_b, Wm, bm):
    wsnp = _sc_transpose(snp, filters)

    pad = N_PAD - N_NODES
    it = jnp.arange(pad, dtype=jnp.int32)
    ids_p = jnp.concatenate(
        [snp_ids, it % N_SNPS]).reshape(TOT_CHUNKS, ROWS_PER_DMA)
    seg_p = jnp.concatenate(
        [segment_ids, N_GENES + it % PAD_SPREAD]).reshape(TOT_CHUNKS,
                                                          ROWS_PER_DMA)

    parts = _sc_kernel(wsnp, ids_p, seg_p)
    p3 = parts.reshape(NC, G_ACC * B // 128, 128)

    W1r = jnp.pad(jnp.transpose(W1.reshape(-1, N_GENES // 8, 8), (2, 0, 1)),
                  ((0, 0), (0, 0), (0, G_ACC // 8 - N_GENES // 8)))
    Wm_p = jnp.concatenate([Wm, jnp.zeros((1, 1), jnp.float32)], axis=1)
    logits = _mlp(p3, W1r,
                  b1.reshape(-1, 1), bn1_w.reshape(-1, 1), bn1_b.reshape(-1, 1),
                  W2,
                  b2.reshape(-1, 1), bn2_w.reshape(-1, 1), bn2_b.reshape(-1, 1),
                  Wm_p, bm.reshape(1, 1))
    return logits.reshape(B, 1)
